# baseline (speedup 1.0000x reference)
import math

import jax
import jax.numpy as jnp
from jax.experimental import pallas as pl

D = 64
B = 16384
_HALF = D // 2


def _fused_kernel(x_ref, c_ref, t_ref, Wstep_ref, bstep_ref,
                  W1_ref, b1_ref, W2_ref, b2_ref, out_ref):
    f32 = jnp.float32
    bp = x_ref.shape[0]

    W1a = W1_ref[0:D, :]
    W1b = W1_ref[D:2 * D, :]
    W1c = W1_ref[2 * D:3 * D, :]
    Wc = jnp.dot(Wstep_ref[:], W1c, preferred_element_type=f32)
    bc = jnp.dot(bstep_ref[:], W1c, preferred_element_type=f32)
    z = jnp.zeros((D, 3 * D), f32)

    def dbl(w):
        top = jnp.concatenate([w, z], axis=1)
        bot = jnp.concatenate([z, w], axis=1)
        return jnp.concatenate([top, bot], axis=0)

    W1a_d = dbl(W1a)
    W1b_d = dbl(W1b)
    Wc_d = dbl(Wc)
    z2 = jnp.zeros((3 * D, D), f32)
    W2_d = jnp.concatenate([
        jnp.concatenate([W2_ref[:], z2], axis=1),
        jnp.concatenate([z2, W2_ref[:]], axis=1),
    ], axis=0)
    b1_row = b1_ref[:] + bc
    b1_d = jnp.concatenate([b1_row, b1_row], axis=1)
    b2_d = jnp.concatenate([b2_ref[:], b2_ref[:]], axis=1)

    t_even = jnp.broadcast_to(t_ref[0:1, :], (D, bp))
    t_odd = jnp.broadcast_to(t_ref[1:2, :], (D, bp))
    tboth = jnp.concatenate([t_even, t_odd], axis=0)
    row = jax.lax.broadcasted_iota(jnp.int32, (2 * D, 1), 0)
    fr = jnp.bitwise_and(row, D - 1)
    fidx = jnp.where(fr < _HALF, fr, fr - _HALF).astype(f32)
    freq_col = jnp.exp(fidx * (-math.log(10000.0) / _HALF))
    shift_col = jnp.where(fr < _HALF, 0.0, math.pi / 2).astype(f32)
    args = freq_col * tboth - shift_col
    n = jnp.round(args * f32(1.0 / (2.0 * math.pi)))
    r = args - n * f32(6.28125)
    r = r - n * f32(1.9353071795864769e-03)
    s = r * r
    tT = f32(-2.2398469402767916e-07)
    tT = tT * s + f32(2.430807671139143e-05)
    tT = tT * s + f32(-1.3867885560937686e-03)
    tT = tT * s + f32(4.1662991555676473e-02)
    tT = tT * s + f32(-4.999981914909368e-01)
    tT = tT * s + f32(1.0)

    h = jnp.dot(x_ref[:], W1a_d, preferred_element_type=f32)
    h += jnp.dot(c_ref[:], W1b_d, preferred_element_type=f32)
    h += jax.lax.dot_general(tT, Wc_d, (((0,), (0,)), ((), ())),
                             preferred_element_type=f32)
    h += b1_d
    h = jnp.where(h > 0, h, 0.01 * h)

    out = jnp.dot(h, W2_d, preferred_element_type=f32)
    out_ref[:] = out + b2_d


@jax.jit
def kernel(x, t, c, W_step, b_step, W1, b1, W2, b2):
    bx = 2048
    bp = bx // 2
    grid = (B // bx,)

    x2 = x.reshape(B // 2, 2 * D)
    c2 = c.reshape(B // 2, 2 * D)
    t_eo = t.astype(jnp.float32).reshape(B // 2, 2).T

    out = pl.pallas_call(
        _fused_kernel,
        grid=grid,
        in_specs=[
            pl.BlockSpec((bp, 2 * D), lambda i: (i, 0)),
            pl.BlockSpec((bp, 2 * D), lambda i: (i, 0)),
            pl.BlockSpec((2, bp), lambda i: (0, i)),
            pl.BlockSpec((D, D), lambda i: (0, 0)),
            pl.BlockSpec((1, D), lambda i: (0, 0)),
            pl.BlockSpec((3 * D, 3 * D), lambda i: (0, 0)),
            pl.BlockSpec((1, 3 * D), lambda i: (0, 0)),
            pl.BlockSpec((3 * D, D), lambda i: (0, 0)),
            pl.BlockSpec((1, D), lambda i: (0, 0)),
        ],
        out_specs=pl.BlockSpec((bp, 2 * D), lambda i: (i, 0)),
        out_shape=jax.ShapeDtypeStruct((B // 2, 2 * D), jnp.float32),
    )(x2, c2, t_eo, W_step, b_step.reshape(1, D),
      W1, b1.reshape(1, 3 * D), W2, b2.reshape(1, D))
    return out.reshape(B, D)


# speedup vs baseline: 3.2389x; 3.2389x over previous
import math

import jax
import jax.numpy as jnp
from jax.experimental import pallas as pl

D = 64
B = 16384
_HALF = D // 2

_TA = (((0,), (0,)), ((), ()))


def _fused_kernel(xT_ref, cT_ref, t_ref, Wstep_ref, W1_ref, W2_ref,
                  bstep_ref, b1_ref, b2_ref, outT_ref):
    f32 = jnp.float32
    bx = t_ref.shape[1]

    W1a = W1_ref[0:D, :]
    W1b = W1_ref[D:2 * D, :]
    W1c = W1_ref[2 * D:3 * D, :]
    Wc = jnp.dot(Wstep_ref[:], W1c, preferred_element_type=f32)
    bc = jax.lax.dot_general(W1c, bstep_ref[:], (((0,), (1,)), ((), ())),
                             preferred_element_type=f32)

    t_row = t_ref[:]
    row = jax.lax.broadcasted_iota(jnp.int32, (D, 1), 0)
    fidx = jnp.where(row < _HALF, row, row - _HALF).astype(f32)
    freq_col = jnp.exp(fidx * (-math.log(10000.0) / _HALF))
    shift_col = jnp.where(row < _HALF, 0.0, math.pi / 2).astype(f32)
    args = freq_col * t_row - shift_col
    n = jnp.round(args * f32(1.0 / (2.0 * math.pi)))
    r = args - n * f32(6.28125)
    r = r - n * f32(1.9353071795864769e-03)
    s = r * r
    tT = f32(-2.2398469402767916e-07)
    tT = tT * s + f32(2.430807671139143e-05)
    tT = tT * s + f32(-1.3867885560937686e-03)
    tT = tT * s + f32(4.1662991555676473e-02)
    tT = tT * s + f32(-4.999981914909368e-01)
    tT = tT * s + f32(1.0)

    h = jax.lax.dot_general(W1a, xT_ref[:], _TA, preferred_element_type=f32)
    h += jax.lax.dot_general(W1b, cT_ref[:], _TA, preferred_element_type=f32)
    h += jax.lax.dot_general(Wc, tT, _TA, preferred_element_type=f32)
    h += b1_ref[:] + bc
    h = jnp.where(h > 0, h, 0.01 * h)

    o = jax.lax.dot_general(W2_ref[:], h, _TA, preferred_element_type=f32)
    outT_ref[:] = o + b2_ref[:]


@jax.jit
def kernel(x, t, c, W_step, b_step, W1, b1, W2, b2):
    bx = 2048
    grid = (B // bx,)

    xT = x.T
    cT = c.T
    tf = t.astype(jnp.float32).reshape(1, B)

    outT = pl.pallas_call(
        _fused_kernel,
        grid=grid,
        in_specs=[
            pl.BlockSpec((D, bx), lambda i: (0, i)),
            pl.BlockSpec((D, bx), lambda i: (0, i)),
            pl.BlockSpec((1, bx), lambda i: (0, i)),
            pl.BlockSpec((D, D), lambda i: (0, 0)),
            pl.BlockSpec((3 * D, 3 * D), lambda i: (0, 0)),
            pl.BlockSpec((3 * D, D), lambda i: (0, 0)),
            pl.BlockSpec((1, D), lambda i: (0, 0)),
            pl.BlockSpec((3 * D, 1), lambda i: (0, 0)),
            pl.BlockSpec((D, 1), lambda i: (0, 0)),
        ],
        out_specs=pl.BlockSpec((D, bx), lambda i: (0, i)),
        out_shape=jax.ShapeDtypeStruct((D, B), jnp.float32),
    )(xT, cT, tf, W_step, W1, W2,
      b_step.reshape(1, D), b1.reshape(3 * D, 1), b2.reshape(D, 1))
    return outT.T


# speedup vs baseline: 4.3007x; 1.3278x over previous
import math

import jax
import jax.numpy as jnp
from jax.experimental import pallas as pl

D = 64
B = 16384
_HALF = D // 2

_TA = (((0,), (0,)), ((), ()))


def _fused_kernel(xT_ref, cT_ref, t_ref, Wstep_ref, W1_ref, W2T_ref,
                  bstep_ref, b1_ref, b2_ref, outT_ref):
    f32 = jnp.float32

    W1a = W1_ref[0:D, :]
    W1b = W1_ref[D:2 * D, :]
    W1c = W1_ref[2 * D:3 * D, :]
    Wc = jnp.dot(Wstep_ref[:], W1c, preferred_element_type=f32)
    bc = jax.lax.dot_general(W1c, bstep_ref[:], (((0,), (1,)), ((), ())),
                             preferred_element_type=f32)
    ones1 = jnp.ones((1, 1), f32)
    b1_col = jax.lax.dot_general(b1_ref[:], ones1, _TA,
                                 preferred_element_type=f32)
    b2_col = jax.lax.dot_general(b2_ref[:], ones1, _TA,
                                 preferred_element_type=f32)

    t_row = t_ref[:].astype(f32)
    row = jax.lax.broadcasted_iota(jnp.int32, (D, 1), 0)
    fidx = jnp.where(row < _HALF, row, row - _HALF).astype(f32)
    freq_col = jnp.exp(fidx * (-math.log(10000.0) / _HALF))
    shift_col = jnp.where(row < _HALF, 0.0, math.pi / 2).astype(f32)
    args = freq_col * t_row - shift_col
    n = jnp.round(args * f32(1.0 / (2.0 * math.pi)))
    r = args - n * f32(6.28125)
    r = r - n * f32(1.9353071795864769e-03)
    s = r * r
    tT = f32(-2.2398469402767916e-07)
    tT = tT * s + f32(2.430807671139143e-05)
    tT = tT * s + f32(-1.3867885560937686e-03)
    tT = tT * s + f32(4.1662991555676473e-02)
    tT = tT * s + f32(-4.999981914909368e-01)
    tT = tT * s + f32(1.0)

    h = jax.lax.dot_general(W1a, xT_ref[:], _TA, preferred_element_type=f32)
    h += jax.lax.dot_general(W1b, cT_ref[:], _TA, preferred_element_type=f32)
    h += jax.lax.dot_general(Wc, tT, _TA, preferred_element_type=f32)
    h += b1_col + bc
    h = jnp.where(h > 0, h, 0.01 * h)

    o = jnp.dot(W2T_ref[:], h, preferred_element_type=f32)
    outT_ref[:] = o + b2_col


@jax.jit
def kernel(x, t, c, W_step, b_step, W1, b1, W2, b2):
    bx = 2048
    grid = (B // bx,)

    xT = x.T
    cT = c.T
    W2T = W2.T
    t2 = t.reshape(1, B)

    outT = pl.pallas_call(
        _fused_kernel,
        grid=grid,
        in_specs=[
            pl.BlockSpec((D, bx), lambda i: (0, i)),
            pl.BlockSpec((D, bx), lambda i: (0, i)),
            pl.BlockSpec((1, bx), lambda i: (0, i)),
            pl.BlockSpec((D, D), lambda i: (0, 0)),
            pl.BlockSpec((3 * D, 3 * D), lambda i: (0, 0)),
            pl.BlockSpec((D, 3 * D), lambda i: (0, 0)),
            pl.BlockSpec((1, D), lambda i: (0, 0)),
            pl.BlockSpec((1, 3 * D), lambda i: (0, 0)),
            pl.BlockSpec((1, D), lambda i: (0, 0)),
        ],
        out_specs=pl.BlockSpec((D, bx), lambda i: (0, i)),
        out_shape=jax.ShapeDtypeStruct((D, B), jnp.float32),
    )(xT, cT, t2, W_step, W1, W2T,
      b_step.reshape(1, D), b1.reshape(1, 3 * D), b2.reshape(1, D))
    return outT.T


# speedup vs baseline: 4.5301x; 1.0534x over previous
import math

import jax
import jax.numpy as jnp
from jax.experimental import pallas as pl

D = 64
B = 16384
_HALF = D // 2

_TA = (((0,), (0,)), ((), ()))


def _fused_kernel(xT_ref, cT_ref, t_ref, Wstep_ref, W1_ref, W2T_ref,
                  bstep_ref, b1_ref, b2_ref, outT_ref):
    f32 = jnp.float32

    W1a = W1_ref[0:D, :]
    W1b = W1_ref[D:2 * D, :]
    W1c = W1_ref[2 * D:3 * D, :]
    Wc = jnp.dot(Wstep_ref[:], W1c, preferred_element_type=f32)
    bc = jax.lax.dot_general(W1c, bstep_ref[:], (((0,), (1,)), ((), ())),
                             preferred_element_type=f32)
    ones1 = jnp.ones((1, 1), f32)
    b1_col = jax.lax.dot_general(b1_ref[:], ones1, _TA,
                                 preferred_element_type=f32)
    b2_col = jax.lax.dot_general(b2_ref[:], ones1, _TA,
                                 preferred_element_type=f32)

    t_row = t_ref[:].astype(f32)
    row = jax.lax.broadcasted_iota(jnp.int32, (D, 1), 0)
    fidx = jnp.where(row < _HALF, row, row - _HALF).astype(f32)
    freq_col = jnp.exp(fidx * (-math.log(10000.0) / _HALF))
    shift_col = jnp.where(row < _HALF, 0.0, math.pi / 2).astype(f32)
    args = freq_col * t_row - shift_col
    n = jnp.round(args * f32(1.0 / (2.0 * math.pi)))
    r = args - n * f32(6.28125)
    r = r - n * f32(1.9353071795864769e-03)
    s = r * r
    tT = f32(-2.2398469402767916e-07)
    tT = tT * s + f32(2.430807671139143e-05)
    tT = tT * s + f32(-1.3867885560937686e-03)
    tT = tT * s + f32(4.1662991555676473e-02)
    tT = tT * s + f32(-4.999981914909368e-01)
    tT = tT * s + f32(1.0)

    h = jax.lax.dot_general(W1a, xT_ref[:], _TA, preferred_element_type=f32)
    h += jax.lax.dot_general(W1b, cT_ref[:], _TA, preferred_element_type=f32)
    h += jax.lax.dot_general(Wc, tT, _TA, preferred_element_type=f32)
    h += b1_col + bc
    h = jnp.where(h > 0, h, 0.01 * h)

    o = jnp.dot(W2T_ref[:], h, preferred_element_type=f32)
    outT_ref[:] = o + b2_col


@jax.jit
def kernel(x, t, c, W_step, b_step, W1, b1, W2, b2):
    bx = 4096
    grid = (B // bx,)

    xT = x.T
    cT = c.T
    W2T = W2.T
    t2 = t.reshape(1, B)

    outT = pl.pallas_call(
        _fused_kernel,
        grid=grid,
        in_specs=[
            pl.BlockSpec((D, bx), lambda i: (0, i)),
            pl.BlockSpec((D, bx), lambda i: (0, i)),
            pl.BlockSpec((1, bx), lambda i: (0, i)),
            pl.BlockSpec((D, D), lambda i: (0, 0)),
            pl.BlockSpec((3 * D, 3 * D), lambda i: (0, 0)),
            pl.BlockSpec((D, 3 * D), lambda i: (0, 0)),
            pl.BlockSpec((1, D), lambda i: (0, 0)),
            pl.BlockSpec((1, 3 * D), lambda i: (0, 0)),
            pl.BlockSpec((1, D), lambda i: (0, 0)),
        ],
        out_specs=pl.BlockSpec((D, bx), lambda i: (0, i)),
        out_shape=jax.ShapeDtypeStruct((D, B), jnp.float32),
    )(xT, cT, t2, W_step, W1, W2T,
      b_step.reshape(1, D), b1.reshape(1, 3 * D), b2.reshape(1, D))
    return outT.T
